# baseline (speedup 1.0000x reference)
import jax
import jax.numpy as jnp
from jax.experimental import pallas as pl
from jax.experimental.pallas import tpu as pltpu

B = 16
C = 256
BC = B * C
S_TOTAL = 5440
S_TILE = 512


def _kern(f0, f1, f2, f3, emb, out_ref):
    i = pl.program_id(0)

    @pl.when(i < 8)
    def _():
        out_ref[...] = f0[...].T + emb[0][None, :]

    @pl.when((i >= 8) & (i < 10))
    def _():
        out_ref[...] = f1[...].T + emb[1][None, :]

    @pl.when(i >= 10)
    def _():
        b = i - 10
        cols = pl.ds(b * C, C)
        out_ref[0:256, cols] = f2[...].reshape(C, 256).T + emb[2][None, 0:C]
        out_ref[256:320, cols] = f3[...].reshape(C, 64).T + emb[3][None, 0:C]


def kernel(feats_0, feats_1, feats_2, feats_3, level_start_idx, spatial_shapes, embed_weight):
    f0 = feats_0.reshape(BC, 4096)
    f1 = feats_1.reshape(BC, 1024)
    emb_bc = jnp.tile(embed_weight, (1, B))
    in_specs = [
        pl.BlockSpec((BC, 512), lambda i: (0, jnp.clip(i, 0, 7))),
        pl.BlockSpec((BC, 512), lambda i: (0, jnp.clip(i - 8, 0, 1))),
        pl.BlockSpec((1, C, 16, 16), lambda i: (jnp.clip(i - 10, 0, B - 1), 0, 0, 0)),
        pl.BlockSpec((1, C, 8, 8), lambda i: (jnp.clip(i - 10, 0, B - 1), 0, 0, 0)),
        pl.BlockSpec((4, BC), lambda i: (0, 0)),
    ]
    out = pl.pallas_call(
        _kern,
        grid=(26,),
        in_specs=in_specs,
        out_specs=pl.BlockSpec((S_TILE, BC), lambda i: (jnp.minimum(i, 10), 0)),
        out_shape=jax.ShapeDtypeStruct((S_TOTAL, BC), jnp.float32),
        compiler_params=pltpu.CompilerParams(
            dimension_semantics=("arbitrary",),
            vmem_limit_bytes=110 * 1024 * 1024,
        ),
    )(f0, f1, feats_2, feats_3, emb_bc)
    return out.reshape(S_TOTAL, B, C)


# speedup vs baseline: 1.1072x; 1.1072x over previous
import jax
import jax.numpy as jnp
from jax.experimental import pallas as pl
from jax.experimental.pallas import tpu as pltpu

B = 16
C = 256
BC = B * C
LEVEL_HW = (4096, 1024, 256, 64)
S_TOTAL = 5440
S_TILE = 512


def _kern(f0, f1, f2, f3, emb, out_ref):
    i = pl.program_id(0)

    @pl.when(i < 8)
    def _():
        out_ref[...] = f0[...].T + emb[0][None, :]

    @pl.when((i >= 8) & (i < 10))
    def _():
        out_ref[...] = f1[...].T + emb[1][None, :]

    @pl.when(i == 10)
    def _():
        out_ref[0:256, :] = f2[...].T + emb[2][None, :]
        out_ref[256:320, :] = f3[...].T + emb[3][None, :]


def kernel(feats_0, feats_1, feats_2, feats_3, level_start_idx, spatial_shapes, embed_weight):
    feats = [
        f.reshape(BC, hw)
        for f, hw in zip((feats_0, feats_1, feats_2, feats_3), LEVEL_HW)
    ]
    emb_bc = jnp.tile(embed_weight, (1, B))
    in_specs = [
        pl.BlockSpec((BC, 512), lambda i: (0, jnp.clip(i, 0, 7))),
        pl.BlockSpec((BC, 512), lambda i: (0, jnp.clip(i - 8, 0, 1))),
        pl.BlockSpec((BC, 256), lambda i: (0, 0)),
        pl.BlockSpec((BC, 64), lambda i: (0, 0)),
        pl.BlockSpec((4, BC), lambda i: (0, 0)),
    ]
    out = pl.pallas_call(
        _kern,
        grid=(11,),
        in_specs=in_specs,
        out_specs=pl.BlockSpec((S_TILE, BC), lambda i: (i, 0)),
        out_shape=jax.ShapeDtypeStruct((S_TOTAL, BC), jnp.float32),
        compiler_params=pltpu.CompilerParams(
            dimension_semantics=("parallel",),
            vmem_limit_bytes=110 * 1024 * 1024,
        ),
    )(*feats, emb_bc)
    return out.reshape(S_TOTAL, B, C)


# speedup vs baseline: 1.1079x; 1.0006x over previous
import jax
import jax.numpy as jnp
from jax.experimental import pallas as pl
from jax.experimental.pallas import tpu as pltpu

B = 16
C = 256
BC = B * C
S_TOTAL = 5440
S_TILE = 512


def _kern0(f0, emb, out_ref):
    out_ref[...] = f0[...].T + emb[0][None, :]


def _kern1(f1, f2, f3, emb, prev, out_ref):
    i = pl.program_id(0)

    @pl.when(i < 2)
    def _():
        out_ref[...] = f1[...].T + emb[1][None, :]

    @pl.when(i == 2)
    def _():
        out_ref[0:256, :] = f2[...].T + emb[2][None, :]
        out_ref[256:320, :] = f3[...].T + emb[3][None, :]


def kernel(feats_0, feats_1, feats_2, feats_3, level_start_idx, spatial_shapes, embed_weight):
    f0 = feats_0.reshape(BC, 4096)
    f1 = feats_1.reshape(BC, 1024)
    f2 = feats_2.reshape(BC, 256)
    f3 = feats_3.reshape(BC, 64)
    emb_bc = jnp.tile(embed_weight, (1, B))
    part = pl.pallas_call(
        _kern0,
        grid=(8,),
        in_specs=[
            pl.BlockSpec((BC, S_TILE), lambda i: (0, i)),
            pl.BlockSpec((4, BC), lambda i: (0, 0)),
        ],
        out_specs=pl.BlockSpec((S_TILE, BC), lambda i: (i, 0)),
        out_shape=jax.ShapeDtypeStruct((S_TOTAL, BC), jnp.float32),
        compiler_params=pltpu.CompilerParams(
            dimension_semantics=("parallel",),
            vmem_limit_bytes=110 * 1024 * 1024,
        ),
    )(f0, emb_bc)
    out = pl.pallas_call(
        _kern1,
        grid=(3,),
        in_specs=[
            pl.BlockSpec((BC, S_TILE), lambda i: (0, jnp.clip(i, 0, 1))),
            pl.BlockSpec((BC, 256), lambda i: (0, 0)),
            pl.BlockSpec((BC, 64), lambda i: (0, 0)),
            pl.BlockSpec((4, BC), lambda i: (0, 0)),
            pl.BlockSpec((8, 128), lambda i: (0, 0)),
        ],
        out_specs=pl.BlockSpec((S_TILE, BC), lambda i: (i + 8, 0)),
        out_shape=jax.ShapeDtypeStruct((S_TOTAL, BC), jnp.float32),
        input_output_aliases={4: 0},
        compiler_params=pltpu.CompilerParams(
            dimension_semantics=("arbitrary",),
            vmem_limit_bytes=110 * 1024 * 1024,
        ),
    )(f1, f2, f3, emb_bc, part)
    return out.reshape(S_TOTAL, B, C)
